# baseline (speedup 1.0000x reference)
import functools

import jax
import jax.numpy as jnp
from jax import lax
from jax.experimental import pallas as pl
from jax.experimental.pallas import tpu as pltpu
from jax.experimental.pallas import tpu_sc as plsc

NC = 2
NS = 16
NW = NC * NS


def _sigmoid(x):
    e = jnp.exp(-jnp.abs(x))
    return jnp.where(x >= 0, 1.0 / (1.0 + e), e / (1.0 + e))


def _tanh(x):
    e = jnp.exp(-2.0 * jnp.abs(x))
    t = (1.0 - e) / (1.0 + e)
    return jnp.where(x >= 0, t, -t)


@jax.jit
def kernel(inputs, state, gate_kernel_users, gate_kernel_items, gate_bias,
           candidate_kernel_users, candidate_kernel_items, candidate_bias):
    B, H = state.shape
    BPW = B // NW
    EPW = BPW * H
    u_idx = inputs[:, 0].astype(jnp.int32)
    i_idx = inputs[:, 1].astype(jnp.int32)
    U1 = gate_kernel_users.shape[0]
    gku2 = gate_kernel_users.reshape(U1, H * 2 * H)
    cku2 = candidate_kernel_users.reshape(U1, H * H)
    ar = jnp.arange(H, dtype=jnp.int32)
    gidx = (i_idx[:, None] * (2 * H) + (H + ar)[None, :]).reshape(B * H // 128, 128)
    cidx = (i_idx[:, None] * H + ar[None, :]).reshape(B * H // 128, 128)
    gkif = gate_kernel_items.reshape(gate_kernel_items.size)
    ckif = candidate_kernel_items.reshape(candidate_kernel_items.size)
    s1 = state.reshape(B * H)
    NCHUNK = EPW // 128

    mesh = plsc.VectorSubcoreMesh(
        core_axis_name="c", subcore_axis_name="s",
        num_cores=NC, num_subcores=NS)

    @functools.partial(
        pl.kernel,
        out_type=jax.ShapeDtypeStruct((B * H,), jnp.float32),
        mesh=mesh,
        scratch_types=[
            pltpu.VMEM((BPW,), jnp.int32),
            pltpu.VMEM((NCHUNK, 128), jnp.int32),
            pltpu.VMEM((NCHUNK, 128), jnp.int32),
            pltpu.VMEM((EPW,), jnp.float32),
            pltpu.VMEM((BPW, H * 2 * H), jnp.float32),
            pltpu.VMEM((NCHUNK, 128), jnp.float32),
            pltpu.VMEM((BPW, H * H), jnp.float32),
            pltpu.VMEM((NCHUNK, 128), jnp.float32),
            pltpu.VMEM((2 * H,), jnp.float32),
            pltpu.VMEM((H,), jnp.float32),
            pltpu.VMEM((EPW,), jnp.float32),
            pltpu.SemaphoreType.DMA,
            pltpu.SemaphoreType.DMA,
            pltpu.SemaphoreType.DMA,
            pltpu.SemaphoreType.DMA,
        ],
    )
    def run(u_hbm, gidx_hbm, cidx_hbm, s_hbm, gku_hbm, gkif_hbm, cku_hbm,
            ckif_hbm, gb_hbm, cb_hbm, out_hbm,
            u_v, gx_v, cx_v, s_v, wg_v, gi_v, wc_v, ci_v, gb_v, cb_v, o_v,
            sem0, sem1, sem2, sem3):
        wid = lax.axis_index("s") * NC + lax.axis_index("c")
        base = wid * BPW
        pltpu.sync_copy(u_hbm.at[pl.ds(base, BPW)], u_v)
        cp0 = pltpu.async_copy(gku_hbm.at[u_v], wg_v, sem0)
        cp1 = pltpu.async_copy(cku_hbm.at[u_v], wc_v, sem1)
        pltpu.sync_copy(gidx_hbm.at[pl.ds(wid * NCHUNK, NCHUNK)], gx_v)
        pltpu.sync_copy(cidx_hbm.at[pl.ds(wid * NCHUNK, NCHUNK)], cx_v)
        item_cps = []
        for j in range(NCHUNK):
            item_cps.append(
                pltpu.async_copy(gkif_hbm.at[gx_v.at[j]], gi_v.at[j], sem2))
            item_cps.append(
                pltpu.async_copy(ckif_hbm.at[cx_v.at[j]], ci_v.at[j], sem3))
        pltpu.sync_copy(s_hbm.at[pl.ds(base * H, EPW)], s_v)
        pltpu.sync_copy(gb_hbm, gb_v)
        pltpu.sync_copy(cb_hbm, cb_v)
        for cp in item_cps:
            cp.wait()
        cp0.wait()
        cp1.wait()

        gbias_hi = gb_v[pl.ds(H, H)]
        cbias = cb_v[...]

        def elem(b, carry):
            j = b // 8
            col = (b % 8) * H
            acc_u = gbias_hi + gi_v[j, pl.ds(col, H)]
            acc_c = cbias + ci_v[j, pl.ds(col, H)]
            sb = s_v[pl.ds(b * H, H)]
            for h in range(H):
                sh = sb[h]
                acc_u = acc_u + sh * wg_v[b, pl.ds(h * 2 * H + H, H)]
                acc_c = acc_c + sh * wc_v[b, pl.ds(h * H, H)]
            u_gate = _sigmoid(acc_u)
            c = _tanh(acc_c)
            o_v[pl.ds(b * H, H)] = u_gate * sb + (1.0 - u_gate) * c
            return carry

        lax.fori_loop(0, BPW, elem, 0)
        pltpu.sync_copy(o_v, out_hbm.at[pl.ds(base * H, EPW)])

    out = run(u_idx, gidx, cidx, s1, gku2, gkif, cku2, ckif,
              gate_bias, candidate_bias)
    return out.reshape(B, H)
